# baseline (speedup 1.0000x reference)
import jax
import jax.numpy as jnp
from jax import lax
from jax.experimental import pallas as pl

Q, K, D, DF = 1024, 100000, 128, 64
KPAD = 100352
QBLK, KBLK = 256, 512


def _l2norm(x, axis):
    n = jnp.linalg.norm(x, axis=axis, keepdims=True)
    return x / jnp.clip(n, 1e-12)


def _w_body(stn_ref, memn_ref, w_ref):
    j = pl.program_id(0)
    sim = lax.dot_general(stn_ref[...], memn_ref[...], (((1,), (1,)), ((), ())),
                          preferred_element_type=jnp.float32)
    col = j * KBLK + lax.broadcasted_iota(jnp.int32, (QBLK, KBLK), 1)
    w_ref[...] = jnp.where(col < K, sim, -2.0)


def kernel(state_past, memory_past, memory_fut, W_q, W_m):
    stn = _l2norm(state_past, axis=1)
    pn = jnp.pad(_l2norm(memory_past, axis=1), ((0, KPAD - K), (0, 0)))

    w = pl.pallas_call(
        _w_body,
        grid=(KPAD // KBLK, Q // QBLK),
        in_specs=[
            pl.BlockSpec((QBLK, D), lambda j, i: (i, 0)),
            pl.BlockSpec((KBLK, D), lambda j, i: (j, 0)),
        ],
        out_specs=pl.BlockSpec((QBLK, KBLK), lambda j, i: (i, j)),
        out_shape=jax.ShapeDtypeStruct((Q, KPAD), jnp.float32),
    )(stn, pn)

    _, index_max = lax.top_k(w, 200)
    mem_p = jnp.take(memory_past, index_max, axis=0)
    mem_f = jnp.take(memory_fut, index_max, axis=0)
    q_sel = _l2norm(state_past @ W_q, axis=1)[:, None, :]
    m_sel = _l2norm(mem_p @ W_m, axis=2)
    w2 = jnp.einsum('qod,qkd->qk', q_sel, m_sel)
    _, idx2 = lax.top_k(w2, 120)
    feat_fut = jnp.take_along_axis(mem_f, idx2[:, :, None], axis=1)
    return feat_fut


# speedup vs baseline: 4.6542x; 4.6542x over previous
import functools

import jax
import jax.numpy as jnp
from jax import lax
from jax.experimental import pallas as pl
from jax.experimental.pallas import tpu as pltpu
from jax.experimental.pallas import tpu_sc as plsc

Q, K, D, DF = 1024, 100000, 128, 64
KPAD = 100352
QBLK, KBLK = 256, 512

NBIN = 4112
CAP = 4096
TOPN = 200
OUTW = 208
L = 16


def _l2norm(x, axis):
    n = jnp.linalg.norm(x, axis=axis, keepdims=True)
    return x / jnp.clip(n, 1e-12)


def _w_body(stn_ref, memn_ref, w_ref):
    j = pl.program_id(0)
    sim = lax.dot_general(stn_ref[...], memn_ref[...], (((1,), (1,)), ((), ())),
                          preferred_element_type=jnp.float32)
    col = j * KBLK + lax.broadcasted_iota(jnp.int32, (QBLK, KBLK), 1)
    w_ref[...] = jnp.where(col < K, sim, -2.0)


def _bin16(v):
    a = (v + 1.0) * 2048.0
    return jnp.clip(a.astype(jnp.int32), 0, NBIN - 9)


def _topk_sc_body(w_hbm, out_hbm, row_v, hist_v, cval_v, cidx_v, outb_v):
    wid = lax.axis_index("s") * 2 + lax.axis_index("c")
    lanes = jnp.arange(L, dtype=jnp.int32)
    ones = jnp.ones((L,), jnp.int32)
    NV = KPAD // L

    def per_query(qq, _):
        q = wid * 32 + qq
        pltpu.sync_copy(w_hbm.at[pl.ds(q * KPAD, KPAD)], row_v)

        def zb(g, _):
            hist_v[pl.ds(g * L, L)] = jnp.zeros((L,), jnp.int32)
            return 0
        lax.fori_loop(0, NBIN // L, zb, 0)

        def h1(i, _):
            v = row_v[pl.ds(i * L, L)]
            plsc.addupdate_scatter(hist_v, [_bin16(v)], ones)
            return 0
        lax.fori_loop(0, NV, h1, 0)

        def scan(g, carry):
            cum, bstar, nge, found = carry
            gg = (NBIN // L - 1) - g
            grp = hist_v[pl.ds(gg * L, L)]
            suf = lax.rev(plsc.cumsum(lax.rev(grp, (0,))), (0,))
            tot = suf + cum
            sel = jnp.where(tot >= TOPN, lanes, -1)
            mx = jnp.max(sel)
            here = mx >= 0
            s_mx = jnp.max(jnp.where(lanes == mx, tot, 0))
            take = jnp.logical_and(here, found == 0)
            bstar = jnp.where(take, gg * L + mx, bstar)
            nge = jnp.where(take, s_mx, nge)
            found = jnp.where(here, 1, found)
            return cum + jnp.sum(grp), bstar, nge, found
        _, bstar, nge, _ = lax.fori_loop(
            0, NBIN // L, scan,
            (jnp.int32(0), jnp.int32(0), jnp.int32(0), jnp.int32(0)))

        ncand = jnp.minimum(nge, CAP)
        nvec = (ncand + (L - 1)) // L

        def c2(i, ptr):
            v = row_v[pl.ds(i * L, L)]
            msk = _bin16(v) >= bstar
            mi = msk.astype(jnp.int32)
            pos = jnp.minimum(ptr + plsc.cumsum(mi) - 1, CAP - 1)
            plsc.store_scatter(cval_v, [pos], v, mask=msk)
            plsc.store_scatter(cidx_v, [pos], i * L + lanes, mask=msk)
            return ptr + jnp.sum(mi)
        lax.fori_loop(0, NV, c2, jnp.int32(0))

        tail = (nvec - 1) * L
        slot = tail + lanes
        tv = cval_v[pl.ds(tail, L)]
        ti = cidx_v[pl.ds(tail, L)]
        cval_v[pl.ds(tail, L)] = jnp.where(slot >= ncand, -3.0, tv)
        cidx_v[pl.ds(tail, L)] = jnp.where(slot >= ncand, jnp.int32(2**31 - 1), ti)

        def ex(s, outvec):
            def rmax(j, m):
                return jnp.maximum(m, cval_v[pl.ds(j * L, L)])
            m = jnp.max(lax.fori_loop(0, nvec, rmax, jnp.full((L,), -4.0)))

            def rimin(j, im):
                v = cval_v[pl.ds(j * L, L)]
                ix = cidx_v[pl.ds(j * L, L)]
                return jnp.minimum(im, jnp.min(jnp.where(v == m, ix,
                                                         jnp.int32(2**31 - 1))))
            imin = lax.fori_loop(0, nvec, rimin, jnp.int32(2**31 - 1))

            def rkill(j, _):
                v = cval_v[pl.ds(j * L, L)]
                ix = cidx_v[pl.ds(j * L, L)]
                hit = jnp.logical_and(v == m, ix == imin)
                cval_v[pl.ds(j * L, L)] = jnp.where(hit, -3.5, v)
                return 0
            lax.fori_loop(0, nvec, rkill, 0)

            outvec = jnp.where(lanes == (s % L), imin, outvec)
            outb_v[pl.ds((s // L) * L, L)] = outvec
            return outvec
        lax.fori_loop(0, OUTW, ex, jnp.zeros((L,), jnp.int32))

        pltpu.sync_copy(outb_v, out_hbm.at[pl.ds(q * OUTW, OUTW)])
        return 0

    lax.fori_loop(0, 32, per_query, 0)


def _topk_sc(w):
    mesh = plsc.VectorSubcoreMesh(core_axis_name="c", subcore_axis_name="s")
    fn = functools.partial(
        pl.kernel,
        mesh=mesh,
        compiler_params=pltpu.CompilerParams(needs_layout_passes=False),
        out_type=jax.ShapeDtypeStruct((Q * OUTW,), jnp.int32),
        scratch_types=[
            pltpu.VMEM((KPAD,), jnp.float32),
            pltpu.VMEM((NBIN,), jnp.int32),
            pltpu.VMEM((CAP,), jnp.float32),
            pltpu.VMEM((CAP,), jnp.int32),
            pltpu.VMEM((OUTW,), jnp.int32),
        ],
    )(_topk_sc_body)
    return fn(w.reshape(-1))


def kernel(state_past, memory_past, memory_fut, W_q, W_m):
    stn = _l2norm(state_past, axis=1)
    pn = jnp.pad(_l2norm(memory_past, axis=1), ((0, KPAD - K), (0, 0)))

    w = pl.pallas_call(
        _w_body,
        grid=(KPAD // KBLK, Q // QBLK),
        in_specs=[
            pl.BlockSpec((QBLK, D), lambda j, i: (i, 0)),
            pl.BlockSpec((KBLK, D), lambda j, i: (j, 0)),
        ],
        out_specs=pl.BlockSpec((QBLK, KBLK), lambda j, i: (i, j)),
        out_shape=jax.ShapeDtypeStruct((Q, KPAD), jnp.float32),
    )(stn, pn)

    index_max = _topk_sc(w).reshape(Q, OUTW)[:, :TOPN]

    mem_p = jnp.take(memory_past, index_max, axis=0)
    mem_f = jnp.take(memory_fut, index_max, axis=0)
    q_sel = _l2norm(state_past @ W_q, axis=1)[:, None, :]
    m_sel = _l2norm(mem_p @ W_m, axis=2)
    w2 = jnp.einsum('qod,qkd->qk', q_sel, m_sel)
    _, idx2 = lax.top_k(w2, 120)
    feat_fut = jnp.take_along_axis(mem_f, idx2[:, :, None], axis=1)
    return feat_fut


# speedup vs baseline: 4.8345x; 1.0387x over previous
import functools

import jax
import jax.numpy as jnp
from jax import lax
from jax.experimental import pallas as pl
from jax.experimental.pallas import tpu as pltpu
from jax.experimental.pallas import tpu_sc as plsc

Q, K, D, DF = 1024, 100000, 128, 64
KPAD = 100352
QBLK, KBLK = 256, 512

NBIN = 4112
CAP = 4096
TOPN = 200
OUTW = 208
L = 16


def _l2norm(x, axis):
    n = jnp.linalg.norm(x, axis=axis, keepdims=True)
    return x / jnp.clip(n, 1e-12)


def _w_body(stn_ref, memn_ref, w_ref):
    j = pl.program_id(0)
    sim = lax.dot_general(stn_ref[...], memn_ref[...], (((1,), (1,)), ((), ())),
                          preferred_element_type=jnp.float32)
    col = j * KBLK + lax.broadcasted_iota(jnp.int32, (QBLK, KBLK), 1)
    w_ref[...] = jnp.where(col < K, sim, -2.0)


def _bin16(v):
    a = (v + 1.0) * 2048.0
    return jnp.clip(a.astype(jnp.int32), 0, NBIN - 9)


def _topk_sc_body(w_hbm, out_hbm, row_v, hist_v, cval_v, cidx_v, outb_v):
    wid = lax.axis_index("s") * 2 + lax.axis_index("c")
    lanes = jnp.arange(L, dtype=jnp.int32)
    ones = jnp.ones((L,), jnp.int32)
    NV = KPAD // L

    def per_query(qq, _):
        q = wid * 32 + qq
        pltpu.sync_copy(w_hbm.at[pl.ds(q * KPAD, KPAD)], row_v)

        def zb(g, _):
            hist_v[pl.ds(g * L, L)] = jnp.zeros((L,), jnp.int32)
            return 0
        lax.fori_loop(0, NBIN // L, zb, 0)

        def h1(i, _):
            base = i * (8 * L)
            for u in range(8):
                v = row_v[pl.ds(base + u * L, L)]
                plsc.addupdate_scatter(hist_v, [_bin16(v)], ones)
            return 0
        lax.fori_loop(0, NV // 8, h1, 0)

        def scan(g, carry):
            cum, bstar, nge, found = carry
            gg = (NBIN // L - 1) - g
            grp = hist_v[pl.ds(gg * L, L)]
            suf = lax.rev(plsc.cumsum(lax.rev(grp, (0,))), (0,))
            tot = suf + cum
            sel = jnp.where(tot >= TOPN, lanes, -1)
            mx = jnp.max(sel)
            here = mx >= 0
            s_mx = jnp.max(jnp.where(lanes == mx, tot, 0))
            take = jnp.logical_and(here, found == 0)
            bstar = jnp.where(take, gg * L + mx, bstar)
            nge = jnp.where(take, s_mx, nge)
            found = jnp.where(here, 1, found)
            return cum + jnp.sum(grp), bstar, nge, found
        _, bstar, nge, _ = lax.fori_loop(
            0, NBIN // L, scan,
            (jnp.int32(0), jnp.int32(0), jnp.int32(0), jnp.int32(0)))

        ncand = jnp.minimum(nge, CAP)
        nvec = (ncand + (L - 1)) // L

        def c2(i, ptr):
            for u in range(4):
                off = i * (4 * L) + u * L
                v = row_v[pl.ds(off, L)]
                msk = _bin16(v) >= bstar
                mi = msk.astype(jnp.int32)
                pos = jnp.minimum(ptr + plsc.cumsum(mi) - 1, CAP - 1)
                plsc.store_scatter(cval_v, [pos], v, mask=msk)
                plsc.store_scatter(cidx_v, [pos], off + lanes, mask=msk)
                ptr = ptr + jnp.sum(mi)
            return ptr
        lax.fori_loop(0, NV // 4, c2, jnp.int32(0))

        tail = (nvec - 1) * L
        slot = tail + lanes
        tv = cval_v[pl.ds(tail, L)]
        ti = cidx_v[pl.ds(tail, L)]
        cval_v[pl.ds(tail, L)] = jnp.where(slot >= ncand, -3.0, tv)
        cidx_v[pl.ds(tail, L)] = jnp.where(slot >= ncand, jnp.int32(2**31 - 1), ti)

        def ex(s, outvec):
            def rmax(j, carry):
                vm, im = carry
                v = cval_v[pl.ds(j * L, L)]
                ix = cidx_v[pl.ds(j * L, L)]
                gt = v > vm
                eq = v == vm
                vm2 = jnp.where(gt, v, vm)
                im2 = jnp.where(gt, ix, jnp.where(eq, jnp.minimum(im, ix), im))
                return vm2, im2
            vm, im = lax.fori_loop(
                0, nvec, rmax,
                (jnp.full((L,), -4.0), jnp.full((L,), 2**31 - 1, jnp.int32)))
            m = jnp.max(vm)
            imin = jnp.min(jnp.where(vm == m, im, jnp.int32(2**31 - 1)))

            def rkill(j, _):
                v = cval_v[pl.ds(j * L, L)]
                ix = cidx_v[pl.ds(j * L, L)]
                hit = jnp.logical_and(v == m, ix == imin)
                cval_v[pl.ds(j * L, L)] = jnp.where(hit, -3.5, v)
                return 0
            lax.fori_loop(0, nvec, rkill, 0)

            outvec = jnp.where(lanes == (s % L), imin, outvec)
            outb_v[pl.ds((s // L) * L, L)] = outvec
            return outvec
        lax.fori_loop(0, OUTW, ex, jnp.zeros((L,), jnp.int32))

        pltpu.sync_copy(outb_v, out_hbm.at[pl.ds(q * OUTW, OUTW)])
        return 0

    lax.fori_loop(0, 32, per_query, 0)


def _topk_sc(w):
    mesh = plsc.VectorSubcoreMesh(core_axis_name="c", subcore_axis_name="s")
    fn = functools.partial(
        pl.kernel,
        mesh=mesh,
        compiler_params=pltpu.CompilerParams(needs_layout_passes=False),
        out_type=jax.ShapeDtypeStruct((Q * OUTW,), jnp.int32),
        scratch_types=[
            pltpu.VMEM((KPAD,), jnp.float32),
            pltpu.VMEM((NBIN,), jnp.int32),
            pltpu.VMEM((CAP,), jnp.float32),
            pltpu.VMEM((CAP,), jnp.int32),
            pltpu.VMEM((OUTW,), jnp.int32),
        ],
    )(_topk_sc_body)
    return fn(w.reshape(-1))


def kernel(state_past, memory_past, memory_fut, W_q, W_m):
    stn = _l2norm(state_past, axis=1)
    pn = jnp.pad(_l2norm(memory_past, axis=1), ((0, KPAD - K), (0, 0)))

    w = pl.pallas_call(
        _w_body,
        grid=(KPAD // KBLK, Q // QBLK),
        in_specs=[
            pl.BlockSpec((QBLK, D), lambda j, i: (i, 0)),
            pl.BlockSpec((KBLK, D), lambda j, i: (j, 0)),
        ],
        out_specs=pl.BlockSpec((QBLK, KBLK), lambda j, i: (i, j)),
        out_shape=jax.ShapeDtypeStruct((Q, KPAD), jnp.float32),
    )(stn, pn)

    index_max = _topk_sc(w).reshape(Q, OUTW)[:, :TOPN]

    mem_p = jnp.take(memory_past, index_max, axis=0)
    mem_f = jnp.take(memory_fut, index_max, axis=0)
    q_sel = _l2norm(state_past @ W_q, axis=1)[:, None, :]
    m_sel = _l2norm(mem_p @ W_m, axis=2)
    w2 = jnp.einsum('qod,qkd->qk', q_sel, m_sel)
    _, idx2 = lax.top_k(w2, 120)
    feat_fut = jnp.take_along_axis(mem_f, idx2[:, :, None], axis=1)
    return feat_fut


# speedup vs baseline: 5.1421x; 1.0636x over previous
import functools

import jax
import jax.numpy as jnp
from jax import lax
from jax.experimental import pallas as pl
from jax.experimental.pallas import tpu as pltpu
from jax.experimental.pallas import tpu_sc as plsc

Q, K, D, DF = 1024, 100000, 128, 64
KPAD = 100352
QBLK, KBLK = 256, 512

NBIN = 4112
CAP = 4096
TOPN = 200
OUTW = 208
L = 16


def _l2norm(x, axis):
    n = jnp.linalg.norm(x, axis=axis, keepdims=True)
    return x / jnp.clip(n, 1e-12)


def _w_body(stn_ref, memn_ref, w_ref):
    j = pl.program_id(0)
    sim = lax.dot_general(stn_ref[...], memn_ref[...], (((1,), (1,)), ((), ())),
                          preferred_element_type=jnp.float32)
    col = j * KBLK + lax.broadcasted_iota(jnp.int32, (QBLK, KBLK), 1)
    w_ref[...] = jnp.where(col < K, sim, -2.0)


def _bin16(v):
    a = (v + 1.0) * 2048.0
    return jnp.clip(a.astype(jnp.int32), 0, NBIN - 9)


def _topk_sc_body(w_hbm, out_hbm, row_v, hist_v, cval_v, cidx_v, outb_v):
    wid = lax.axis_index("s") * 2 + lax.axis_index("c")
    lanes = jnp.arange(L, dtype=jnp.int32)
    ones = jnp.ones((L,), jnp.int32)
    NV = KPAD // L

    def per_query(qq, _):
        q = wid * 32 + qq
        pltpu.sync_copy(w_hbm.at[pl.ds(q * KPAD, KPAD)], row_v)

        def zb(g, _):
            hist_v[pl.ds(g * L, L)] = jnp.zeros((L,), jnp.int32)
            return 0
        lax.fori_loop(0, NBIN // L, zb, 0)

        def h1(i, _):
            base = i * (8 * L)
            for u in range(8):
                v = row_v[pl.ds(base + u * L, L)]
                plsc.addupdate_scatter(hist_v, [_bin16(v)], ones)
            return 0
        lax.fori_loop(0, NV // 8, h1, 0)

        def scan(g, carry):
            cum, bstar, nge, found = carry
            gg = (NBIN // L - 1) - g
            grp = hist_v[pl.ds(gg * L, L)]
            suf = lax.rev(plsc.cumsum(lax.rev(grp, (0,))), (0,))
            tot = suf + cum
            sel = jnp.where(tot >= TOPN, lanes, -1)
            mx = jnp.max(sel)
            here = mx >= 0
            s_mx = jnp.max(jnp.where(lanes == mx, tot, 0))
            take = jnp.logical_and(here, found == 0)
            bstar = jnp.where(take, gg * L + mx, bstar)
            nge = jnp.where(take, s_mx, nge)
            found = jnp.where(here, 1, found)
            return cum + jnp.sum(grp), bstar, nge, found
        _, bstar, nge, _ = lax.fori_loop(
            0, NBIN // L, scan,
            (jnp.int32(0), jnp.int32(0), jnp.int32(0), jnp.int32(0)))

        ncand = jnp.minimum(nge, CAP)
        nvec = (ncand + (L - 1)) // L

        def c2(i, ptr):
            for u in range(4):
                off = i * (4 * L) + u * L
                v = row_v[pl.ds(off, L)]
                msk = _bin16(v) >= bstar
                mi = msk.astype(jnp.int32)
                pos = jnp.minimum(ptr + plsc.cumsum(mi) - 1, CAP - 1)
                plsc.store_scatter(cval_v, [pos], v, mask=msk)
                plsc.store_scatter(cidx_v, [pos], off + lanes, mask=msk)
                ptr = ptr + jnp.sum(mi)
            return ptr
        lax.fori_loop(0, NV // 4, c2, jnp.int32(0))

        tail = (nvec - 1) * L
        slot = tail + lanes
        tv = cval_v[pl.ds(tail, L)]
        ti = cidx_v[pl.ds(tail, L)]
        cval_v[pl.ds(tail, L)] = jnp.where(slot >= ncand, -3.0, tv)
        cidx_v[pl.ds(tail, L)] = jnp.where(slot >= ncand, jnp.int32(2**31 - 1), ti)

        def ex(s, carry):
            outvec, mp, ip = carry

            def rmax(j, c):
                vm, im = c
                v = cval_v[pl.ds(j * L, L)]
                ix = cidx_v[pl.ds(j * L, L)]
                elig = jnp.logical_or(
                    v < mp, jnp.logical_and(v == mp, ix > ip))
                gt = jnp.logical_and(elig, v > vm)
                eq = jnp.logical_and(elig, v == vm)
                vm2 = jnp.where(gt, v, vm)
                im2 = jnp.where(gt, ix, jnp.where(eq, jnp.minimum(im, ix), im))
                return vm2, im2
            vm, im = lax.fori_loop(
                0, nvec, rmax,
                (jnp.full((L,), -4.0), jnp.full((L,), 2**31 - 1, jnp.int32)))
            m = jnp.max(vm)
            imin = jnp.min(jnp.where(vm == m, im, jnp.int32(2**31 - 1)))

            outvec = jnp.where(lanes == (s % L), imin, outvec)
            outb_v[pl.ds((s // L) * L, L)] = outvec
            return outvec, m, imin
        lax.fori_loop(0, OUTW, ex,
                      (jnp.zeros((L,), jnp.int32), jnp.float32(1e9),
                       jnp.int32(-1)))

        pltpu.sync_copy(outb_v, out_hbm.at[pl.ds(q * OUTW, OUTW)])
        return 0

    lax.fori_loop(0, 32, per_query, 0)


def _topk_sc(w):
    mesh = plsc.VectorSubcoreMesh(core_axis_name="c", subcore_axis_name="s")
    fn = functools.partial(
        pl.kernel,
        mesh=mesh,
        compiler_params=pltpu.CompilerParams(needs_layout_passes=False),
        out_type=jax.ShapeDtypeStruct((Q * OUTW,), jnp.int32),
        scratch_types=[
            pltpu.VMEM((KPAD,), jnp.float32),
            pltpu.VMEM((NBIN,), jnp.int32),
            pltpu.VMEM((CAP,), jnp.float32),
            pltpu.VMEM((CAP,), jnp.int32),
            pltpu.VMEM((OUTW,), jnp.int32),
        ],
    )(_topk_sc_body)
    return fn(w.reshape(-1))


def kernel(state_past, memory_past, memory_fut, W_q, W_m):
    stn = _l2norm(state_past, axis=1)
    pn = jnp.pad(_l2norm(memory_past, axis=1), ((0, KPAD - K), (0, 0)))

    w = pl.pallas_call(
        _w_body,
        grid=(KPAD // KBLK, Q // QBLK),
        in_specs=[
            pl.BlockSpec((QBLK, D), lambda j, i: (i, 0)),
            pl.BlockSpec((KBLK, D), lambda j, i: (j, 0)),
        ],
        out_specs=pl.BlockSpec((QBLK, KBLK), lambda j, i: (i, j)),
        out_shape=jax.ShapeDtypeStruct((Q, KPAD), jnp.float32),
    )(stn, pn)

    index_max = _topk_sc(w).reshape(Q, OUTW)[:, :TOPN]

    mem_p = jnp.take(memory_past, index_max, axis=0)
    mem_f = jnp.take(memory_fut, index_max, axis=0)
    q_sel = _l2norm(state_past @ W_q, axis=1)[:, None, :]
    m_sel = _l2norm(mem_p @ W_m, axis=2)
    w2 = jnp.einsum('qod,qkd->qk', q_sel, m_sel)
    _, idx2 = lax.top_k(w2, 120)
    feat_fut = jnp.take_along_axis(mem_f, idx2[:, :, None], axis=1)
    return feat_fut


# speedup vs baseline: 6.9929x; 1.3599x over previous
import functools

import jax
import jax.numpy as jnp
from jax import lax
from jax.experimental import pallas as pl
from jax.experimental.pallas import tpu as pltpu
from jax.experimental.pallas import tpu_sc as plsc

Q, K, D, DF = 1024, 100000, 128, 64
KPAD = 100352
QBLK, KBLK = 256, 512

NBIN = 4112
CAP = 4096
TOPN = 200
OUTW = 208
L = 16


def _l2norm(x, axis):
    n = jnp.linalg.norm(x, axis=axis, keepdims=True)
    return x / jnp.clip(n, 1e-12)


def _w_body(stn_ref, memn_ref, w_ref):
    j = pl.program_id(0)
    sim = lax.dot_general(stn_ref[...], memn_ref[...], (((1,), (1,)), ((), ())),
                          preferred_element_type=jnp.float32)
    col = j * KBLK + lax.broadcasted_iota(jnp.int32, (QBLK, KBLK), 1)
    w_ref[...] = jnp.where(col < K, sim, -2.0)


def _bin16(v):
    a = (v + 1.0) * 2048.0
    return jnp.clip(a.astype(jnp.int32), 0, NBIN - 9)


def _topk_sc_body(w_hbm, out_hbm, row_v, hist_v, cval_v, cidx_v, outb_v):
    wid = lax.axis_index("s") * 2 + lax.axis_index("c")
    lanes = jnp.arange(L, dtype=jnp.int32)
    ones = jnp.ones((L,), jnp.int32)
    NV = KPAD // L

    def per_query(qq, _):
        q = wid * 32 + qq
        pltpu.sync_copy(w_hbm.at[pl.ds(q * KPAD, KPAD)], row_v)

        def zb(g, _):
            hist_v[pl.ds(g * L, L)] = jnp.zeros((L,), jnp.int32)
            return 0
        lax.fori_loop(0, NBIN // L, zb, 0)

        def h1(i, _):
            base = i * (8 * L)
            for u in range(8):
                v = row_v[pl.ds(base + u * L, L)]
                plsc.addupdate_scatter(hist_v, [_bin16(v)], ones)
            return 0
        lax.fori_loop(0, NV // 8, h1, 0)

        def scan(g, carry):
            cum, bstar, nge, found = carry
            gg = (NBIN // L - 1) - g
            grp = hist_v[pl.ds(gg * L, L)]
            suf = lax.rev(plsc.cumsum(lax.rev(grp, (0,))), (0,))
            tot = suf + cum
            sel = jnp.where(tot >= TOPN, lanes, -1)
            mx = jnp.max(sel)
            here = mx >= 0
            s_mx = jnp.max(jnp.where(lanes == mx, tot, 0))
            take = jnp.logical_and(here, found == 0)
            bstar = jnp.where(take, gg * L + mx, bstar)
            nge = jnp.where(take, s_mx, nge)
            found = jnp.where(here, 1, found)
            return cum + jnp.sum(grp), bstar, nge, found
        _, bstar, nge, _ = lax.fori_loop(
            0, NBIN // L, scan,
            (jnp.int32(0), jnp.int32(0), jnp.int32(0), jnp.int32(0)))

        ncand = jnp.minimum(nge, CAP)
        nvec = (ncand + (L - 1)) // L

        def c2(i, ptrv):
            vs, msks, css, pcs = [], [], [], []
            for u in range(4):
                off = i * (4 * L) + u * L
                v = row_v[pl.ds(off, L)]
                msk = _bin16(v) >= bstar
                vs.append(v)
                msks.append(msk)
                css.append(plsc.cumsum(msk.astype(jnp.int32)))
                pcs.append(plsc.all_reduce_population_count(msk))
            for u in range(4):
                off = i * (4 * L) + u * L
                pos = jnp.minimum(ptrv + css[u] - 1, CAP - 1)
                plsc.store_scatter(cval_v, [pos], vs[u], mask=msks[u])
                plsc.store_scatter(cidx_v, [pos], off + lanes, mask=msks[u])
                ptrv = ptrv + pcs[u]
            return ptrv
        ptrv = lax.fori_loop(0, NV // 4, c2, jnp.zeros((L,), jnp.int32))

        tail = (nvec - 1) * L
        slot = tail + lanes
        tv = cval_v[pl.ds(tail, L)]
        ti = cidx_v[pl.ds(tail, L)]
        cval_v[pl.ds(tail, L)] = jnp.where(slot >= ncand, -3.0, tv)
        cidx_v[pl.ds(tail, L)] = jnp.where(slot >= ncand, jnp.int32(2**31 - 1), ti)

        def ex(s, carry):
            outvec, mp, ip = carry

            def rmax(j, c):
                vm, im = c
                v = cval_v[pl.ds(j * L, L)]
                ix = cidx_v[pl.ds(j * L, L)]
                elig = jnp.logical_or(
                    v < mp, jnp.logical_and(v == mp, ix > ip))
                gt = jnp.logical_and(elig, v > vm)
                eq = jnp.logical_and(elig, v == vm)
                vm2 = jnp.where(gt, v, vm)
                im2 = jnp.where(gt, ix, jnp.where(eq, jnp.minimum(im, ix), im))
                return vm2, im2
            vm, im = lax.fori_loop(
                0, nvec, rmax,
                (jnp.full((L,), -4.0), jnp.full((L,), 2**31 - 1, jnp.int32)))
            m = jnp.max(vm)
            imin = jnp.min(jnp.where(vm == m, im, jnp.int32(2**31 - 1)))

            outvec = jnp.where(lanes == (s % L), imin, outvec)
            outb_v[pl.ds((s // L) * L, L)] = outvec
            return outvec, m, imin
        lax.fori_loop(0, OUTW, ex,
                      (jnp.zeros((L,), jnp.int32), jnp.float32(1e9),
                       jnp.int32(-1)))

        pltpu.sync_copy(outb_v, out_hbm.at[pl.ds(q * OUTW, OUTW)])
        return 0

    lax.fori_loop(0, 32, per_query, 0)


def _topk_sc(w):
    mesh = plsc.VectorSubcoreMesh(core_axis_name="c", subcore_axis_name="s")
    fn = functools.partial(
        pl.kernel,
        mesh=mesh,
        compiler_params=pltpu.CompilerParams(needs_layout_passes=False),
        out_type=jax.ShapeDtypeStruct((Q * OUTW,), jnp.int32),
        scratch_types=[
            pltpu.VMEM((KPAD,), jnp.float32),
            pltpu.VMEM((NBIN,), jnp.int32),
            pltpu.VMEM((CAP,), jnp.float32),
            pltpu.VMEM((CAP,), jnp.int32),
            pltpu.VMEM((OUTW,), jnp.int32),
        ],
    )(_topk_sc_body)
    return fn(w.reshape(-1))


def kernel(state_past, memory_past, memory_fut, W_q, W_m):
    stn = _l2norm(state_past, axis=1)
    pn = jnp.pad(_l2norm(memory_past, axis=1), ((0, KPAD - K), (0, 0)))

    w = pl.pallas_call(
        _w_body,
        grid=(KPAD // KBLK, Q // QBLK),
        in_specs=[
            pl.BlockSpec((QBLK, D), lambda j, i: (i, 0)),
            pl.BlockSpec((KBLK, D), lambda j, i: (j, 0)),
        ],
        out_specs=pl.BlockSpec((QBLK, KBLK), lambda j, i: (i, j)),
        out_shape=jax.ShapeDtypeStruct((Q, KPAD), jnp.float32),
    )(stn, pn)

    index_max = _topk_sc(w).reshape(Q, OUTW)[:, :TOPN]

    mem_p = jnp.take(memory_past, index_max, axis=0)
    mem_f = jnp.take(memory_fut, index_max, axis=0)
    q_sel = _l2norm(state_past @ W_q, axis=1)[:, None, :]
    m_sel = _l2norm(mem_p @ W_m, axis=2)
    w2 = jnp.einsum('qod,qkd->qk', q_sel, m_sel)
    _, idx2 = lax.top_k(w2, 120)
    feat_fut = jnp.take_along_axis(mem_f, idx2[:, :, None], axis=1)
    return feat_fut


# speedup vs baseline: 7.0017x; 1.0013x over previous
import functools

import jax
import jax.numpy as jnp
from jax import lax
from jax.experimental import pallas as pl
from jax.experimental.pallas import tpu as pltpu
from jax.experimental.pallas import tpu_sc as plsc

Q, K, D, DF = 1024, 100000, 128, 64
KPAD = 100352
QBLK, KBLK = 256, 512

NBIN = 4112
CAP = 4096
TOPN = 200
OUTW = 208
L = 16


def _l2norm(x, axis):
    n = jnp.linalg.norm(x, axis=axis, keepdims=True)
    return x / jnp.clip(n, 1e-12)


def _w_body(stn_ref, memn_ref, w_ref):
    j = pl.program_id(0)
    sim = lax.dot_general(stn_ref[...], memn_ref[...], (((1,), (1,)), ((), ())),
                          preferred_element_type=jnp.float32)
    col = j * KBLK + lax.broadcasted_iota(jnp.int32, (QBLK, KBLK), 1)
    w_ref[...] = jnp.where(col < K, sim, -2.0)


def _bin16(v):
    a = (v + 1.0) * 2048.0
    return jnp.clip(a.astype(jnp.int32), 0, NBIN - 9)


def _topk_sc_body(w_hbm, out_hbm, row_v, hist_v, cval_v, cidx_v, outb_v):
    wid = lax.axis_index("s") * 2 + lax.axis_index("c")
    lanes = jnp.arange(L, dtype=jnp.int32)
    ones = jnp.ones((L,), jnp.int32)
    NV = KPAD // L

    def per_query(qq, _):
        q = wid * 32 + qq
        pltpu.sync_copy(w_hbm.at[pl.ds(q * KPAD, KPAD)], row_v)

        def zb(g, _):
            hist_v[pl.ds(g * L, L)] = jnp.zeros((L,), jnp.int32)
            return 0
        lax.fori_loop(0, NBIN // L, zb, 0)

        def h1(i, _):
            base = i * (8 * L)
            for u in range(8):
                v = row_v[pl.ds(base + u * L, L)]
                plsc.addupdate_scatter(hist_v, [_bin16(v)], ones)
            return 0
        lax.fori_loop(0, NV // 8, h1, 0)

        def scan(g, carry):
            cum, bstar, nge, found = carry
            gg = (NBIN // L - 1) - g
            grp = hist_v[pl.ds(gg * L, L)]
            suf = lax.rev(plsc.cumsum(lax.rev(grp, (0,))), (0,))
            tot = suf + cum
            sel = jnp.where(tot >= TOPN, lanes, -1)
            mx = jnp.max(sel)
            here = mx >= 0
            s_mx = jnp.max(jnp.where(lanes == mx, tot, 0))
            take = jnp.logical_and(here, found == 0)
            bstar = jnp.where(take, gg * L + mx, bstar)
            nge = jnp.where(take, s_mx, nge)
            found = jnp.where(here, 1, found)
            return cum + jnp.sum(grp), bstar, nge, found
        _, bstar, nge, _ = lax.fori_loop(
            0, NBIN // L, scan,
            (jnp.int32(0), jnp.int32(0), jnp.int32(0), jnp.int32(0)))

        ncand = jnp.minimum(nge, CAP)
        nvec = (ncand + (L - 1)) // L

        def c2(i, ptrv):
            vs, msks, css, pcs = [], [], [], []
            for u in range(4):
                off = i * (4 * L) + u * L
                v = row_v[pl.ds(off, L)]
                msk = _bin16(v) >= bstar
                vs.append(v)
                msks.append(msk)
                css.append(plsc.cumsum(msk.astype(jnp.int32)))
                pcs.append(plsc.all_reduce_population_count(msk))
            for u in range(4):
                off = i * (4 * L) + u * L
                pos = jnp.minimum(ptrv + css[u] - 1, CAP - 1)
                plsc.store_scatter(cval_v, [pos], vs[u], mask=msks[u])
                plsc.store_scatter(cidx_v, [pos], off + lanes, mask=msks[u])
                ptrv = ptrv + pcs[u]
            return ptrv
        ptrv = lax.fori_loop(0, NV // 4, c2, jnp.zeros((L,), jnp.int32))

        tail = (nvec - 1) * L
        slot = tail + lanes
        tv = cval_v[pl.ds(tail, L)]
        ti = cidx_v[pl.ds(tail, L)]
        cval_v[pl.ds(tail, L)] = jnp.where(slot >= ncand, -3.0, tv)
        cidx_v[pl.ds(tail, L)] = jnp.where(slot >= ncand, jnp.int32(2**31 - 1), ti)
        cval_v[pl.ds(nvec * L, L)] = jnp.full((L,), -3.0)
        cidx_v[pl.ds(nvec * L, L)] = jnp.full((L,), 2**31 - 1, jnp.int32)

        def ex(s, carry):
            outvec, mp, ip = carry

            def rmax(j, c):
                vm, im = c
                for u in range(2):
                    off = (2 * j + u) * L
                    v = cval_v[pl.ds(off, L)]
                    ix = cidx_v[pl.ds(off, L)]
                    elig = jnp.logical_or(
                        v < mp, jnp.logical_and(v == mp, ix > ip))
                    gt = jnp.logical_and(elig, v > vm)
                    eq = jnp.logical_and(elig, v == vm)
                    vm = jnp.where(gt, v, vm)
                    im = jnp.where(gt, ix,
                                   jnp.where(eq, jnp.minimum(im, ix), im))
                return vm, im
            vm, im = lax.fori_loop(
                0, (nvec + 1) // 2, rmax,
                (jnp.full((L,), -4.0), jnp.full((L,), 2**31 - 1, jnp.int32)))
            m = jnp.max(vm)
            imin = jnp.min(jnp.where(vm == m, im, jnp.int32(2**31 - 1)))

            outvec = jnp.where(lanes == (s % L), imin, outvec)
            outb_v[pl.ds((s // L) * L, L)] = outvec
            return outvec, m, imin
        lax.fori_loop(0, OUTW, ex,
                      (jnp.zeros((L,), jnp.int32), jnp.float32(1e9),
                       jnp.int32(-1)))

        pltpu.sync_copy(outb_v, out_hbm.at[pl.ds(q * OUTW, OUTW)])
        return 0

    lax.fori_loop(0, 32, per_query, 0)


def _topk_sc(w):
    mesh = plsc.VectorSubcoreMesh(core_axis_name="c", subcore_axis_name="s")
    fn = functools.partial(
        pl.kernel,
        mesh=mesh,
        compiler_params=pltpu.CompilerParams(needs_layout_passes=False),
        out_type=jax.ShapeDtypeStruct((Q * OUTW,), jnp.int32),
        scratch_types=[
            pltpu.VMEM((KPAD,), jnp.float32),
            pltpu.VMEM((NBIN,), jnp.int32),
            pltpu.VMEM((CAP + L,), jnp.float32),
            pltpu.VMEM((CAP + L,), jnp.int32),
            pltpu.VMEM((OUTW,), jnp.int32),
        ],
    )(_topk_sc_body)
    return fn(w.reshape(-1))


def kernel(state_past, memory_past, memory_fut, W_q, W_m):
    stn = _l2norm(state_past, axis=1)
    pn = jnp.pad(_l2norm(memory_past, axis=1), ((0, KPAD - K), (0, 0)))

    w = pl.pallas_call(
        _w_body,
        grid=(KPAD // KBLK, Q // QBLK),
        in_specs=[
            pl.BlockSpec((QBLK, D), lambda j, i: (i, 0)),
            pl.BlockSpec((KBLK, D), lambda j, i: (j, 0)),
        ],
        out_specs=pl.BlockSpec((QBLK, KBLK), lambda j, i: (i, j)),
        out_shape=jax.ShapeDtypeStruct((Q, KPAD), jnp.float32),
    )(stn, pn)

    index_max = _topk_sc(w).reshape(Q, OUTW)[:, :TOPN]

    mem_p = jnp.take(memory_past, index_max, axis=0)
    mem_f = jnp.take(memory_fut, index_max, axis=0)
    q_sel = _l2norm(state_past @ W_q, axis=1)[:, None, :]
    m_sel = _l2norm(mem_p @ W_m, axis=2)
    w2 = jnp.einsum('qod,qkd->qk', q_sel, m_sel)
    _, idx2 = lax.top_k(w2, 120)
    feat_fut = jnp.take_along_axis(mem_f, idx2[:, :, None], axis=1)
    return feat_fut
